# baseline (speedup 1.0000x reference)
import functools

import jax
import jax.numpy as jnp
import numpy as np
from jax import lax
from jax.experimental import pallas as pl
from jax.experimental.pallas import tpu as pltpu
from jax.experimental.pallas import tpu_sc as plsc

SEQ = 32768
NE = 64
NG = 8
GS = NE // NG
NC, NS, L = 2, 16, 16
NW = NC * NS
RW = SEQ // NW
NBLK = RW // L
BR = 2048


def _treemax(vals):
    while len(vals) > 1:
        vals = [jnp.maximum(vals[2 * i], vals[2 * i + 1])
                for i in range(len(vals) // 2)]
    return vals[0]


def _treemin(vals):
    while len(vals) > 1:
        vals = [jnp.minimum(vals[2 * i], vals[2 * i + 1])
                for i in range(len(vals) // 2)]
    return vals[0]


def _treesum(vals):
    while len(vals) > 1:
        vals = [vals[2 * i] + vals[2 * i + 1]
                for i in range(len(vals) // 2)]
    return vals[0]


SEQH = SEQ // 2
BR2 = 2048


_BCAST2 = np.kron(np.eye(2, dtype=np.float32),
                  np.ones((1, NE), np.float32))
_BLOCKDIAG = np.kron(np.eye(2, dtype=np.float32),
                     np.ones((NE, NE), np.float32))


def _softmax_tc_body(x_ref, bc_ref, bd_ref, rw_ref):
    x = x_ref[...]
    ma = jnp.max(x[:, 0:NE], axis=1, keepdims=True)
    mb = jnp.max(x[:, NE:2 * NE], axis=1, keepdims=True)
    m128 = jax.lax.dot_general(
        jnp.concatenate([ma, mb], axis=1), bc_ref[...],
        (((1,), (0,)), ((), ())), preferred_element_type=jnp.float32)
    e = jnp.exp(x - m128)
    s = jax.lax.dot_general(e, bd_ref[...],
                            (((1,), (0,)), ((), ())),
                            preferred_element_type=jnp.float32)
    rw_ref[...] = e / s


_softmax_tc = pl.pallas_call(
    _softmax_tc_body,
    grid=(SEQH // BR2,),
    in_specs=[pl.BlockSpec((BR2, 2 * NE), lambda i: (i, 0)),
              pl.BlockSpec((2, 2 * NE), lambda i: (0, 0)),
              pl.BlockSpec((2 * NE, 2 * NE), lambda i: (0, 0))],
    out_specs=pl.BlockSpec((BR2, 2 * NE), lambda i: (i, 0)),
    out_shape=jax.ShapeDtypeStruct((SEQH, 2 * NE), jnp.float32),
)


def _router_body(p3_hbm, w3_hbm, ids3_hbm, cnt_hbm,
                 in_v, w_v, ids_v, cnt_v, sem_in):
    wid = lax.axis_index("s") * NC + lax.axis_index("c")

    lanes = jnp.arange(L, dtype=jnp.int32)
    zeros_f = jnp.zeros((L,), jnp.float32)
    ones_f = jnp.ones((L,), jnp.float32)

    in_dma = pltpu.async_copy(p3_hbm.at[wid], in_v, sem_in)

    for e in range(NE):
        cnt_v[pl.ds(e * L, L)] = zeros_f

    in_dma.wait()

    def block_body(b):
        r = b * L

        gmax = []
        gidx = []
        for g in range(NG):
            x = [in_v[g * GS + j, pl.ds(r, L)] for j in range(GS)]
            best = _treemax(list(x))
            cand = [jnp.where(x[j] == best,
                              jnp.full((L,), j, jnp.int32),
                              jnp.full((L,), GS, jnp.int32))
                    for j in range(GS)]
            gmax.append(best)
            gidx.append(_treemin(cand))

        tot = _treesum(list(gmax))
        tinv = ones_f / tot

        for g in range(NG):
            w_v[g, pl.ds(r, L)] = gmax[g] * tinv
            gid = gidx[g] + (g * GS)
            ids_v[g, pl.ds(r, L)] = gid
            plsc.addupdate_scatter(cnt_v, [gid * L + lanes], ones_f)

    plsc.parallel_loop(0, NBLK, 1, unroll=2)(block_body)

    pltpu.sync_copy(w_v, w3_hbm.at[wid])
    pltpu.sync_copy(ids_v, ids3_hbm.at[wid])

    acc = [jnp.zeros((L,), jnp.float32) for _ in range(NE // L)]
    for e in range(NE):
        v = cnt_v[pl.ds(e * L, L)]
        sv = jnp.full((L,), jnp.sum(v), jnp.float32)
        q, rr = divmod(e, L)
        acc[q] = jnp.where(lanes == rr, sv, acc[q])
    for q in range(NE // L):
        cnt_v[pl.ds(q * L, L)] = acc[q]
    pltpu.sync_copy(cnt_v.at[pl.ds(0, NE)], cnt_hbm.at[pl.ds(wid * NE, NE)])


_router = functools.partial(
    pl.kernel,
    out_type=[
        jax.ShapeDtypeStruct((NW, NG, RW), jnp.float32),
        jax.ShapeDtypeStruct((NW, NG, RW), jnp.int32),
        jax.ShapeDtypeStruct((NW * NE,), jnp.float32),
    ],
    mesh=plsc.VectorSubcoreMesh(core_axis_name="c", subcore_axis_name="s",
                                num_cores=NC, num_subcores=NS),
    compiler_params=pltpu.CompilerParams(needs_layout_passes=False),
    scratch_types=[
        pltpu.VMEM((NE, RW), jnp.float32),
        pltpu.VMEM((NG, RW), jnp.float32),
        pltpu.VMEM((NG, RW), jnp.int32),
        pltpu.VMEM((NE * L,), jnp.float32),
        pltpu.SemaphoreType.DMA,
    ],
)(_router_body)


@jax.jit
def kernel(logits):
    rw = _softmax_tc(logits.reshape(SEQH, 2 * NE),
                     jnp.asarray(_BCAST2),
                     jnp.asarray(_BLOCKDIAG)).reshape(SEQ, NE)
    p3 = rw.reshape(NW, RW, NE).transpose(0, 2, 1)
    w3, ids3, cnt_part = _router(p3)
    topk_weights = w3.transpose(0, 2, 1).reshape(SEQ, NG)
    topk_ids = ids3.transpose(0, 2, 1).reshape(SEQ, NG)
    tokens_per_expert = cnt_part.reshape(NW, NE).sum(axis=0)
    return (logits, rw, topk_weights, topk_ids, tokens_per_expert)


# speedup vs baseline: 1.4669x; 1.4669x over previous
import functools

import jax
import jax.numpy as jnp
import numpy as np
from jax import lax
from jax.experimental import pallas as pl
from jax.experimental.pallas import tpu as pltpu
from jax.experimental.pallas import tpu_sc as plsc

SEQ = 32768
NE = 64
NG = 8
GS = NE // NG
NC, NS, L = 2, 16, 16
NW = NC * NS
RW = SEQ // NW
NBLK = RW // L


def _treemax(vals):
    while len(vals) > 1:
        vals = [jnp.maximum(vals[2 * i], vals[2 * i + 1])
                for i in range(len(vals) // 2)]
    return vals[0]


def _treemin(vals):
    while len(vals) > 1:
        vals = [jnp.minimum(vals[2 * i], vals[2 * i + 1])
                for i in range(len(vals) // 2)]
    return vals[0]


def _treesum(vals):
    while len(vals) > 1:
        vals = [vals[2 * i] + vals[2 * i + 1]
                for i in range(len(vals) // 2)]
    return vals[0]


def _router_body(x3_hbm, rw3_hbm, w3_hbm, ids3_hbm, cnt_hbm,
                 in_v, w_v, ids_v, cnt_v, sem_in):
    wid = lax.axis_index("s") * NC + lax.axis_index("c")

    lanes = jnp.arange(L, dtype=jnp.int32)
    zeros_f = jnp.zeros((L,), jnp.float32)
    ones_f = jnp.ones((L,), jnp.float32)

    in_dma = pltpu.async_copy(x3_hbm.at[wid], in_v, sem_in)

    for e in range(NE):
        cnt_v[pl.ds(e * L, L)] = zeros_f

    in_dma.wait()

    def block_body(b):
        r = b * L

        gmax = []
        gidx = []
        tg = []
        for g in range(NG):
            x = [in_v[g * GS + j, pl.ds(r, L)] for j in range(GS)]
            best = _treemax(list(x))
            cand = [jnp.where(x[j] == best,
                              jnp.full((L,), j, jnp.int32),
                              jnp.full((L,), GS, jnp.int32))
                    for j in range(GS)]
            bidx = _treemin(cand)
            q = [jnp.exp(x[j] - best) for j in range(GS)]
            for j in range(GS):
                in_v[g * GS + j, pl.ds(r, L)] = q[j]
            tg.append(_treesum(q))
            gmax.append(best)
            gidx.append(bidx)

        m = _treemax(list(gmax))
        pg = [jnp.exp(gmax[g] - m) for g in range(NG)]
        tinv = ones_f / _treesum(list(pg))
        sinv = ones_f / _treesum([pg[g] * tg[g] for g in range(NG)])

        for g in range(NG):
            w_v[g, pl.ds(r, L)] = pg[g] * tinv
            gid = gidx[g] + (g * GS)
            ids_v[g, pl.ds(r, L)] = gid
            plsc.addupdate_scatter(cnt_v, [gid * L + lanes], ones_f)
            fct = pg[g] * sinv
            for j in range(GS):
                e = g * GS + j
                in_v[e, pl.ds(r, L)] = in_v[e, pl.ds(r, L)] * fct

    plsc.parallel_loop(0, NBLK, 1, unroll=2)(block_body)

    pltpu.sync_copy(in_v, rw3_hbm.at[wid])
    pltpu.sync_copy(w_v, w3_hbm.at[wid])
    pltpu.sync_copy(ids_v, ids3_hbm.at[wid])

    acc = [jnp.zeros((L,), jnp.float32) for _ in range(NE // L)]
    for e in range(NE):
        v = cnt_v[pl.ds(e * L, L)]
        sv = jnp.full((L,), jnp.sum(v), jnp.float32)
        q, rr = divmod(e, L)
        acc[q] = jnp.where(lanes == rr, sv, acc[q])
    for q in range(NE // L):
        cnt_v[pl.ds(q * L, L)] = acc[q]
    pltpu.sync_copy(cnt_v.at[pl.ds(0, NE)], cnt_hbm.at[pl.ds(wid * NE, NE)])


_router = functools.partial(
    pl.kernel,
    out_type=[
        jax.ShapeDtypeStruct((NW, NE, RW), jnp.float32),
        jax.ShapeDtypeStruct((NW, NG, RW), jnp.float32),
        jax.ShapeDtypeStruct((NW, NG, RW), jnp.int32),
        jax.ShapeDtypeStruct((NW * NE,), jnp.float32),
    ],
    mesh=plsc.VectorSubcoreMesh(core_axis_name="c", subcore_axis_name="s",
                                num_cores=NC, num_subcores=NS),
    compiler_params=pltpu.CompilerParams(needs_layout_passes=False),
    scratch_types=[
        pltpu.VMEM((NE, RW), jnp.float32),
        pltpu.VMEM((NG, RW), jnp.float32),
        pltpu.VMEM((NG, RW), jnp.int32),
        pltpu.VMEM((NE * L,), jnp.float32),
        pltpu.SemaphoreType.DMA,
    ],
)(_router_body)


@jax.jit
def kernel(logits):
    x3 = logits.reshape(NW, RW, NE).transpose(0, 2, 1)
    rw3, w3, ids3, cnt_part = _router(x3)
    routing_weights = rw3.transpose(0, 2, 1).reshape(SEQ, NE)
    topk_weights = w3.transpose(0, 2, 1).reshape(SEQ, NG)
    topk_ids = ids3.transpose(0, 2, 1).reshape(SEQ, NG)
    tokens_per_expert = cnt_part.reshape(NW, NE).sum(axis=0)
    return (logits, routing_weights, topk_weights, topk_ids, tokens_per_expert)


# speedup vs baseline: 1.9648x; 1.3395x over previous
import functools

import jax
import jax.numpy as jnp
from jax import lax
from jax.experimental import pallas as pl
from jax.experimental.pallas import tpu as pltpu
from jax.experimental.pallas import tpu_sc as plsc

SEQ = 32768
NE = 64
NG = 8
GS = NE // NG
NC, NS, L = 2, 16, 16
NW = NC * NS
ROWS_PER_W = SEQ // NW
CR = 256
NCHUNK = ROWS_PER_W // CR
NBLK = CR // L


def _treemax(vals):
    while len(vals) > 1:
        vals = [jnp.maximum(vals[2 * i], vals[2 * i + 1])
                for i in range(len(vals) // 2)]
    return vals[0]


def _treemin(vals):
    while len(vals) > 1:
        vals = [jnp.minimum(vals[2 * i], vals[2 * i + 1])
                for i in range(len(vals) // 2)]
    return vals[0]


def _treesum(vals):
    while len(vals) > 1:
        vals = [vals[2 * i] + vals[2 * i + 1]
                for i in range(len(vals) // 2)]
    return vals[0]


def _router_body(in_hbm, rw_hbm, w_hbm, ids_hbm, cnt_hbm,
                 in_v, rw_v, w_v, ids_v, cnt_v,
                 sem_in0, sem_in1, sem_out0, sem_out1):
    sem_in = [sem_in0, sem_in1]
    sem_out = [sem_out0, sem_out1]
    wid = lax.axis_index("s") * NC + lax.axis_index("c")
    base = wid * ROWS_PER_W

    lanes = jnp.arange(L, dtype=jnp.int32)
    zeros_f = jnp.zeros((L,), jnp.float32)
    ones_f = jnp.ones((L,), jnp.float32)

    for e in range(NE):
        cnt_v[pl.ds(e * L, L)] = zeros_f

    def make_block_body(ibuf):
        in_b = in_v.at[ibuf]
        rw_b = rw_v.at[ibuf]
        w_b = w_v.at[ibuf]
        ids_b = ids_v.at[ibuf]

        def block_body(b):
            r = b * L

            gmax = []
            gidx = []
            tg = []
            for g in range(NG):
                x = [in_b[g * GS + j, pl.ds(r, L)] for j in range(GS)]
                best = _treemax(list(x))
                cand = [jnp.where(x[j] == best,
                                  jnp.full((L,), j, jnp.int32),
                                  jnp.full((L,), GS, jnp.int32))
                        for j in range(GS)]
                bidx = _treemin(cand)
                q = [jnp.exp(x[j] - best) for j in range(GS)]
                for j in range(GS):
                    rw_b[g * GS + j, pl.ds(r, L)] = q[j]
                t = _treesum(q)
                gmax.append(best)
                gidx.append(bidx)
                tg.append(t)

            m = _treemax(list(gmax))
            pg = [jnp.exp(gmax[g] - m) for g in range(NG)]
            tot = _treesum(list(pg))
            tinv = ones_f / tot
            s = _treesum([pg[g] * tg[g] for g in range(NG)])
            sinv = ones_f / s

            for g in range(NG):
                w_b[g, pl.ds(r, L)] = pg[g] * tinv
                gid = gidx[g] + (g * GS)
                ids_b[g, pl.ds(r, L)] = gid
                plsc.addupdate_scatter(cnt_v, [gid * L + lanes], ones_f)
                fct = pg[g] * sinv
                for j in range(GS):
                    e = g * GS + j
                    rw_b[e, pl.ds(r, L)] = rw_b[e, pl.ds(r, L)] * fct

        return block_body

    def start_in(c):
        row0 = base + c * CR
        return pltpu.async_copy(in_hbm.at[:, pl.ds(row0, CR)],
                                in_v.at[c % 2], sem_in[c % 2])

    in_dma = [start_in(0)]
    out_dma = {}
    for c in range(NCHUNK):
        if c + 1 < NCHUNK:
            in_dma.append(start_in(c + 1))
        in_dma[c].wait()
        if c >= 2:
            for h in out_dma[c - 2]:
                h.wait()
        plsc.parallel_loop(0, NBLK, 1, unroll=2)(make_block_body(c % 2))
        row0 = base + c * CR
        out_dma[c] = [
            pltpu.async_copy(rw_v.at[c % 2], rw_hbm.at[:, pl.ds(row0, CR)],
                             sem_out[c % 2]),
            pltpu.async_copy(w_v.at[c % 2], w_hbm.at[:, pl.ds(row0, CR)],
                             sem_out[c % 2]),
            pltpu.async_copy(ids_v.at[c % 2], ids_hbm.at[:, pl.ds(row0, CR)],
                             sem_out[c % 2]),
        ]
    for c in range(max(0, NCHUNK - 2), NCHUNK):
        for h in out_dma[c]:
            h.wait()

    acc = [jnp.zeros((L,), jnp.float32) for _ in range(NE // L)]
    for e in range(NE):
        v = cnt_v[pl.ds(e * L, L)]
        sv = jnp.full((L,), jnp.sum(v), jnp.float32)
        q, rr = divmod(e, L)
        acc[q] = jnp.where(lanes == rr, sv, acc[q])
    for q in range(NE // L):
        cnt_v[pl.ds(q * L, L)] = acc[q]
    pltpu.sync_copy(cnt_v.at[pl.ds(0, NE)], cnt_hbm.at[pl.ds(wid * NE, NE)])


_router = functools.partial(
    pl.kernel,
    out_type=[
        jax.ShapeDtypeStruct((NE, SEQ), jnp.float32),
        jax.ShapeDtypeStruct((NG, SEQ), jnp.float32),
        jax.ShapeDtypeStruct((NG, SEQ), jnp.int32),
        jax.ShapeDtypeStruct((NW * NE,), jnp.float32),
    ],
    mesh=plsc.VectorSubcoreMesh(core_axis_name="c", subcore_axis_name="s",
                                num_cores=NC, num_subcores=NS),
    compiler_params=pltpu.CompilerParams(needs_layout_passes=False),
    scratch_types=[
        pltpu.VMEM((2, NE, CR), jnp.float32),
        pltpu.VMEM((2, NE, CR), jnp.float32),
        pltpu.VMEM((2, NG, CR), jnp.float32),
        pltpu.VMEM((2, NG, CR), jnp.int32),
        pltpu.VMEM((NE * L,), jnp.float32),
        pltpu.SemaphoreType.DMA,
        pltpu.SemaphoreType.DMA,
        pltpu.SemaphoreType.DMA,
        pltpu.SemaphoreType.DMA,
    ],
)(_router_body)


@jax.jit
def kernel(logits):
    rw_t, w_t, ids_t, cnt_part = _router(logits.T)
    routing_weights = rw_t.T
    topk_weights = w_t.T
    topk_ids = ids_t.T
    tokens_per_expert = cnt_part.reshape(NW, NE).sum(axis=0)
    return (logits, routing_weights, topk_weights, topk_ids, tokens_per_expert)
